# baseline (speedup 1.0000x reference)
import jax
import jax.numpy as jnp
from jax.experimental import pallas as pl
from jax.experimental.pallas import tpu as pltpu

N = 20000
CONF_THRESHOLD = 0.02
NMS_THRESHOLD = 0.4
PRE_NMS_TOPK = 5000
POST_NMS_TOPK = 750
VAR0, VAR1 = 0.1, 0.2
SCALE = 640.0

B = 512
NB = 10
NPAD = NB * B


def _mask_body(s_ref, o_ref):
    s = s_ref[:]
    o_ref[:] = jnp.where(s > CONF_THRESHOLD, s, -jnp.inf)


def _nms_body(pc, ts, keep_ref, bx_ref, lm_ref,
              s_ref, tri_ref, id_ref,
              x1_ref, y1_ref, x2_ref, y2_ref, ar_ref):
    f32 = jnp.float32
    lx = pc[0:NB, :]
    ly = pc[NB:2 * NB, :]
    eww = pc[2 * NB:3 * NB, :]
    ewh = pc[3 * NB:4 * NB, :]
    pcx = pc[4 * NB:5 * NB, :]
    pcy = pc[5 * NB:6 * NB, :]
    pw = pc[6 * NB:7 * NB, :]
    ph = pc[7 * NB:8 * NB, :]
    cx = pcx + lx * VAR0 * pw
    cy = pcy + ly * VAR0 * ph
    sw = pw * eww
    sh = ph * ewh
    x1u = cx - sw / 2.0
    y1u = cy - sh / 2.0
    x1 = x1u * SCALE
    y1 = y1u * SCALE
    x2 = (x1u + sw) * SCALE
    y2 = (y1u + sh) * SCALE
    x1_ref[:] = x1
    y1_ref[:] = y1
    x2_ref[:] = x2
    y2_ref[:] = y2
    ar_ref[:] = jnp.maximum(x2 - x1, 0.0) * jnp.maximum(y2 - y1, 0.0)
    bx_ref[:] = jnp.concatenate([x1, y1, x2, y2], axis=0)

    lmpts = []
    for i in range(5):
        lmx = pc[(8 + 2 * i) * NB:(9 + 2 * i) * NB, :]
        lmy = pc[(9 + 2 * i) * NB:(10 + 2 * i) * NB, :]
        lmpts.append((pcx + lmx * VAR0 * pw) * SCALE)
        lmpts.append((pcy + lmy * VAR0 * ph) * SCALE)
    lm_ref[:] = jnp.concatenate(lmpts, axis=0)

    keep_ref[:] = jnp.where(ts[:] > -jnp.inf, 1.0, 0.0)
    rix = jax.lax.broadcasted_iota(jnp.int32, (B, B), 0)
    cix = jax.lax.broadcasted_iota(jnp.int32, (B, B), 1)
    tri_ref[:] = jnp.where(cix > rix, 1.0, 0.0)
    id_ref[:] = jnp.where(cix == rix, 1.0, 0.0)

    def kb_body(kb, carry):
        v8 = jnp.concatenate(
            [x1_ref[pl.ds(kb, 1), :], y1_ref[pl.ds(kb, 1), :],
             x2_ref[pl.ds(kb, 1), :], y2_ref[pl.ds(kb, 1), :],
             ar_ref[pl.ds(kb, 1), :], jnp.zeros((3, B), f32)], axis=0)
        c = jax.lax.dot_general(id_ref[:], v8, (((1,), (1,)), ((), ())),
                                precision=jax.lax.Precision.HIGHEST,
                                preferred_element_type=f32)
        x1r = c[:, 0:1]
        y1r = c[:, 1:2]
        x2r = c[:, 2:3]
        y2r = c[:, 3:4]
        arr = c[:, 4:5]

        def iou_blk(cb):
            x1c = x1_ref[pl.ds(cb, 1), :]
            y1c = y1_ref[pl.ds(cb, 1), :]
            x2c = x2_ref[pl.ds(cb, 1), :]
            y2c = y2_ref[pl.ds(cb, 1), :]
            arc = ar_ref[pl.ds(cb, 1), :]
            xx1 = jnp.maximum(x1r, x1c)
            yy1 = jnp.maximum(y1r, y1c)
            xx2 = jnp.minimum(x2r, x2c)
            yy2 = jnp.minimum(y2r, y2c)
            inter = jnp.maximum(xx2 - xx1, 0.0) * jnp.maximum(yy2 - yy1, 0.0)
            return inter / (arr + arc - inter + 1e-9)

        s_ref[:] = jnp.where(iou_blk(kb) > NMS_THRESHOLD, 1.0, 0.0) * tri_ref[:]
        kv0 = keep_ref[pl.ds(kb, 1), :]

        def fcond(c):
            return c[0]

        def fbody(c):
            _, kv = c
            k8v = jnp.broadcast_to(kv, (8, B))
            sup = jax.lax.dot_general(k8v, s_ref[:], (((1,), (0,)), ((), ())),
                                      preferred_element_type=f32)
            kvn = jnp.where(sup[0:1, :] > 0.5, 0.0, kv0)
            changed = jnp.sum(jnp.abs(kvn - kv)) > 0.0
            return changed, kvn

        _, kv = jax.lax.while_loop(fcond, fbody, (True, kv0))
        keep_ref[pl.ds(kb, 1), :] = kv
        k8 = jnp.broadcast_to(kv, (8, B))

        def jb_body(jb, c2):
            so = jnp.where(iou_blk(jb) > NMS_THRESHOLD, 1.0, 0.0)
            sup = jax.lax.dot_general(k8, so, (((1,), (0,)), ((), ())),
                                      preferred_element_type=f32)
            kj = keep_ref[pl.ds(jb, 1), :]
            keep_ref[pl.ds(jb, 1), :] = jnp.where(sup[0:1, :] > 0.5, 0.0, kj)
            return c2

        jax.lax.fori_loop(kb + 1, NB, jb_body, 0)
        return carry

    jax.lax.fori_loop(0, NB, kb_body, 0)


def kernel(loc, conf, landmarks, priors):
    f32 = jnp.float32
    scores = conf[:, 1]
    masked = pl.pallas_call(
        _mask_body,
        out_shape=jax.ShapeDtypeStruct((8, N // 8), f32),
    )(scores.reshape(8, N // 8)).reshape(N)

    ts, order = jax.lax.top_k(masked, PRE_NMS_TOPK)

    npad = NPAD - PRE_NMS_TOPK
    loc_s = jnp.concatenate([loc[order], jnp.zeros((npad, 4), f32)])
    pri_s = jnp.concatenate([priors[order], jnp.zeros((npad, 4), f32)])
    lmk_s = jnp.concatenate([landmarks[order], jnp.zeros((npad, 10), f32)])
    ew = jnp.exp(loc_s[:, 2:4] * VAR1)

    fields = jnp.concatenate([loc_s[:, 0:2], ew, pri_s, lmk_s], axis=1)
    pack_c = fields.T.reshape(18 * NB, B)
    tsp = jnp.concatenate([ts, jnp.full((npad,), -jnp.inf, f32)]).reshape(NB, B)

    keep2d, bx, lm = pl.pallas_call(
        _nms_body,
        out_shape=[
            jax.ShapeDtypeStruct((NB, B), f32),
            jax.ShapeDtypeStruct((4 * NB, B), f32),
            jax.ShapeDtypeStruct((10 * NB, B), f32),
        ],
        scratch_shapes=[
            pltpu.VMEM((B, B), f32),
            pltpu.VMEM((B, B), f32),
            pltpu.VMEM((B, B), f32),
            pltpu.VMEM((NB, B), f32),
            pltpu.VMEM((NB, B), f32),
            pltpu.VMEM((NB, B), f32),
            pltpu.VMEM((NB, B), f32),
            pltpu.VMEM((NB, B), f32),
        ],
    )(pack_c, tsp)

    keepb = keep2d.reshape(NPAD)[:PRE_NMS_TOPK] > 0.5
    sel = jnp.where(keepb, ts, -1e30)
    _, keep_order = jax.lax.top_k(sel, POST_NMS_TOPK)
    bxf = bx.reshape(4, NPAD)
    lmf = lm.reshape(10, NPAD)
    det = jnp.take(bxf, keep_order, axis=1).T
    lmg = jnp.take(lmf, keep_order, axis=1).T
    sc = ts[keep_order]
    ks = keepb[keep_order]
    out = jnp.concatenate([det, sc[:, None], lmg], axis=1)
    return jnp.where(ks[:, None], out, 0.0)
